# baseline (speedup 1.0000x reference)
import functools

import jax
import jax.numpy as jnp
from jax import lax
from jax.experimental import pallas as pl
from jax.experimental.pallas import tpu as pltpu
from jax.experimental.pallas import tpu_sc as plsc

_K = 10
_NC = 2
_NS = 16
_NW = _NC * _NS
_F32 = jnp.float32
_I32 = jnp.int32


def _cdiv(a, b):
    return (a + b - 1) // b


def _rup(a, b):
    return _cdiv(a, b) * b


_SC_PARAMS = pltpu.CompilerParams(
    needs_layout_passes=False, use_tc_tiling_on_sc=False)


def _mesh():
    return plsc.VectorSubcoreMesh(core_axis_name="c", subcore_axis_name="s")


@functools.lru_cache(maxsize=None)
def _matmul_call(n, cin, kc, cout, bn):
    def body(x_ref, g_ref, r_ref, b_ref, xg_ref, xr_ref):
        xb = x_ref[...]
        xg_ref[...] = jnp.dot(xb, g_ref[...], preferred_element_type=_F32)
        xr_ref[...] = jnp.dot(xb, r_ref[...], preferred_element_type=_F32) + b_ref[...]

    return pl.pallas_call(
        body,
        grid=(_cdiv(n, bn),),
        in_specs=[
            pl.BlockSpec((bn, cin), lambda i: (i, 0)),
            pl.BlockSpec((cin, kc), lambda i: (0, 0)),
            pl.BlockSpec((cin, cout), lambda i: (0, 0)),
            pl.BlockSpec((1, cout), lambda i: (0, 0)),
        ],
        out_specs=[
            pl.BlockSpec((bn, kc), lambda i: (i, 0)),
            pl.BlockSpec((bn, cout), lambda i: (i, 0)),
        ],
        out_shape=[
            jax.ShapeDtypeStruct((n, kc), _F32),
            jax.ShapeDtypeStruct((n, cout), _F32),
        ],
    )


def _matmul_tc(x, p):
    n, cin = x.shape
    kc = p["g"].shape[1]
    cout = p["root"].shape[1]
    bn = min(1024, _rup(n, 8))
    return _matmul_call(n, cin, kc, cout, bn)(
        x, p["g"], p["root"], p["bias"].reshape(1, cout))


@functools.lru_cache(maxsize=None)
def _gauss_call(e, be):
    def body(p_ref, m_ref, c_ref, w_ref):
        p0 = p_ref[:, 0:1]
        p1 = p_ref[:, 1:2]
        w_ref[...] = jnp.exp((p0 - m_ref[0:1, :]) ** 2 * c_ref[0:1, :]
                             + (p1 - m_ref[1:2, :]) ** 2 * c_ref[1:2, :])

    return pl.pallas_call(
        body,
        grid=(_cdiv(e, be),),
        in_specs=[
            pl.BlockSpec((be, 2), lambda i: (i, 0)),
            pl.BlockSpec((2, 16), lambda i: (0, 0)),
            pl.BlockSpec((2, 16), lambda i: (0, 0)),
        ],
        out_specs=pl.BlockSpec((be, 16), lambda i: (i, 0)),
        out_shape=jax.ShapeDtypeStruct((e, 16), _F32),
    )


def _gauss_w(pseudo, p):
    e = pseudo.shape[0]
    m = jnp.zeros((2, 16), _F32).at[:, :_K].set(p["mu"].T)
    c = jnp.zeros((2, 16), _F32).at[:, :_K].set(
        (-0.5 / (1e-15 + p["sigma"] * p["sigma"])).T)
    be = min(2048, _rup(e, 8))
    return _gauss_call(e, be)(pseudo, m, c)


@functools.lru_cache(maxsize=None)
def _comb1_call(n, cout, bn, relu):
    def body(a0, a1, c0, c1, xr, o):
        cnt = jnp.maximum(c0[...] + c1[...], 1.0)
        r = (a0[...] + a1[...]) / cnt + xr[...]
        o[...] = jnp.maximum(r, 0.0) if relu else r

    s2 = lambda: pl.BlockSpec((bn, cout), lambda i: (i, 0))
    s1 = lambda: pl.BlockSpec((bn, 1), lambda i: (i, 0))
    return pl.pallas_call(
        body,
        grid=(_cdiv(n, bn),),
        in_specs=[s2(), s2(), s1(), s1(), s2()],
        out_specs=s2(),
        out_shape=jax.ShapeDtypeStruct((n, cout), _F32),
    )


@functools.lru_cache(maxsize=None)
def _comb2_call(n, cout, bn, identity_shortcut):
    def body(a20, a21, c0, c1, xr2, s0, s1_, xrs, o):
        cnt = jnp.maximum(c0[...] + c1[...], 1.0)
        h2 = (a20[...] + a21[...]) / cnt + xr2[...]
        if identity_shortcut:
            xs = xrs[...]
        else:
            xs = (s0[...] + s1_[...]) / cnt + xrs[...]
        o[...] = jnp.maximum(h2 + xs, 0.0)

    s2 = lambda: pl.BlockSpec((bn, cout), lambda i: (i, 0))
    s1 = lambda: pl.BlockSpec((bn, 1), lambda i: (i, 0))
    return pl.pallas_call(
        body,
        grid=(_cdiv(n, bn),),
        in_specs=[s2(), s2(), s1(), s1(), s2(), s2(), s2(), s2()],
        out_specs=s2(),
        out_shape=jax.ShapeDtypeStruct((n, cout), _F32),
    )


def _head_tc(feat_x, feat_y):
    def body(fx_ref, fy_ref, o_ref):
        fx = fx_ref[...]
        fy = fy_ref[...]
        nx = jnp.sqrt(jnp.sum(fx * fx, axis=1, keepdims=True))
        ny = jnp.sqrt(jnp.sum(fy * fy, axis=1, keepdims=True))
        o_ref[:, :128] = fx / nx
        o_ref[:, 128:] = fy / ny

    return pl.pallas_call(
        body, out_shape=jax.ShapeDtypeStruct((162, 256), _F32)
    )(feat_x, feat_y)


def _softmax_tc(x):
    def body(x_ref, o_ref):
        v = x_ref[...]
        m = jnp.max(v, axis=1, keepdims=True)
        e = jnp.exp(v - m)
        o_ref[...] = e / jnp.sum(e, axis=1, keepdims=True)

    return pl.pallas_call(
        body, out_shape=jax.ShapeDtypeStruct(x.shape, x.dtype)
    )(x)


@functools.lru_cache(maxsize=None)
def _edge_call(e_pad, n, n1, cout, c_chunk, with_cnt):
    kc = _K * cout
    ncb = cout // 16
    eu = 4 if ncb <= 2 else 1
    per_w = e_pad // _NW
    n_chunks = per_w // c_chunk
    zrows = n1 // _NS
    zb = 64
    nz = zrows // zb

    out_type = [jax.ShapeDtypeStruct((_NC, n1, cout), _F32)]
    scratch = [
        pltpu.VMEM((c_chunk,), _I32),
        pltpu.VMEM((c_chunk,), _I32),
        pltpu.VMEM((c_chunk,), _I32),
        pltpu.VMEM((c_chunk, 16), _F32),
        pltpu.VMEM((c_chunk, kc), _F32),
        pltpu.VMEM((c_chunk, kc), _F32),
        pltpu.VMEM((c_chunk, cout), _F32),
        pltpu.VMEM((zb, cout), _F32),
        pltpu.VMEM_SHARED((n1, cout), _F32),
        pltpu.SemaphoreType.DMA,
        pltpu.SemaphoreType.DMA,
    ]
    if with_cnt:
        out_type.append(jax.ShapeDtypeStruct((_NC, n1), _F32))
        scratch += [
            pltpu.VMEM((_rup(c_chunk, 16),), _F32),
            pltpu.VMEM((zrows,), _F32),
            pltpu.VMEM_SHARED((n1,), _F32),
        ]

    def body(xg, w, src, dst, *rest):
        if with_cnt:
            (out, cnt_out, idx_v0, idx_v1, dst_v, w_v, rows_v0, rows_v1,
             msg_v, zb_v, agg_sh, sem0, sem1, ones_v, cz_v, cnt_sh) = rest
        else:
            (out, idx_v0, idx_v1, dst_v, w_v, rows_v0, rows_v1, msg_v, zb_v,
             agg_sh, sem0, sem1) = rest
        cid = lax.axis_index("c")
        sid = lax.axis_index("s")
        wid = sid * _NC + cid
        iota = lax.iota(_I32, 16)

        def fill_zb(i, _):
            for cb in range(ncb):
                zb_v[i, pl.ds(cb * 16, 16)] = jnp.zeros((16,), _F32)
            return 0
        lax.fori_loop(0, zb, fill_zb, 0)
        if with_cnt:
            def fill_ones(i, _):
                ones_v[pl.ds(i * 16, 16)] = jnp.ones((16,), _F32)
                return 0
            lax.fori_loop(0, _rup(c_chunk, 16) // 16, fill_ones, 0)

            def fill_cz(i, _):
                cz_v[pl.ds(i * 16, 16)] = jnp.zeros((16,), _F32)
                return 0
            lax.fori_loop(0, zrows // 16, fill_cz, 0)

        def zero_sh(j, _):
            pltpu.sync_copy(zb_v, agg_sh.at[pl.ds(sid * zrows + j * zb, zb)])
            return 0
        lax.fori_loop(0, nz, zero_sh, 0)
        if with_cnt:
            pltpu.sync_copy(cz_v, cnt_sh.at[pl.ds(sid * zrows, zrows)])
        plsc.subcore_barrier()

        base = wid * per_w

        def start_gather(j, idx_b, rows_b, sem_b):
            pltpu.sync_copy(src.at[pl.ds(base + j * c_chunk, c_chunk)], idx_b)
            pltpu.make_async_copy(xg.at[idx_b], rows_b, sem_b).start()

        def compute(j, idx_b, rows_b, sem_b):
            b = base + j * c_chunk
            pltpu.sync_copy(dst.at[pl.ds(b, c_chunk)], dst_v)
            pltpu.sync_copy(w.at[pl.ds(b, c_chunk)], w_v)
            pltpu.make_async_copy(xg.at[idx_b], rows_b, sem_b).wait()

            def edge(q, _):
                for u in range(eu):
                    ei = q * eu + u
                    ei16 = jnp.full((16,), ei, _I32)
                    accs = [jnp.zeros((16,), _F32) for _ in range(ncb)]
                    for k in range(_K):
                        wk = plsc.load_gather(
                            w_v, [ei16, jnp.full((16,), k, _I32)])
                        for cb in range(ncb):
                            r = plsc.load_gather(
                                rows_b, [ei16, iota + (k * cout + cb * 16)])
                            accs[cb] = accs[cb] + wk * r
                    for cb in range(ncb):
                        plsc.store_scatter(msg_v, [ei16, iota + cb * 16],
                                           accs[cb])
                return 0
            lax.fori_loop(0, c_chunk // eu, edge, 0)

            pltpu.sync_copy(msg_v, agg_sh.at[dst_v], add=True)
            if with_cnt:
                pltpu.sync_copy(ones_v.at[pl.ds(0, c_chunk)], cnt_sh.at[dst_v],
                                add=True)

        start_gather(0, idx_v0, rows_v0, sem0)

        def pair(jj, _):
            j0 = 2 * jj
            start_gather(j0 + 1, idx_v1, rows_v1, sem1)
            compute(j0, idx_v0, rows_v0, sem0)

            @pl.when(j0 + 2 < n_chunks)
            def _():
                start_gather(j0 + 2, idx_v0, rows_v0, sem0)
            compute(j0 + 1, idx_v1, rows_v1, sem1)
            return 0
        lax.fori_loop(0, n_chunks // 2, pair, 0)
        if n_chunks % 2 == 1:
            compute(n_chunks - 1, idx_v0, rows_v0, sem0)
        plsc.subcore_barrier()

        def drain(j, _):
            r0 = sid * zrows + j * zb
            pltpu.sync_copy(agg_sh.at[pl.ds(r0, zb)], zb_v)
            pltpu.sync_copy(zb_v, out.at[cid, pl.ds(r0, zb)])
            return 0
        lax.fori_loop(0, nz, drain, 0)
        if with_cnt:
            pltpu.sync_copy(cnt_sh.at[pl.ds(sid * zrows, zrows)], cz_v)
            pltpu.sync_copy(cz_v, cnt_out.at[cid, pl.ds(sid * zrows, zrows)])

    return pl.kernel(body, out_type=out_type, scratch_types=scratch,
                     mesh=_mesh(), compiler_params=_SC_PARAMS)


@functools.lru_cache(maxsize=None)
def _ups_call(m_pad, f, c_chunk):
    per_w = m_pad // _NW
    n_chunks = per_w // c_chunk
    fb = f // 16

    scratch = [
        pltpu.VMEM((c_chunk,), _I32),
        pltpu.VMEM((c_chunk,), _I32),
        pltpu.VMEM((c_chunk, f), _F32),
        pltpu.VMEM((c_chunk, f), _F32),
        pltpu.VMEM((c_chunk, f), _F32),
        pltpu.SemaphoreType.DMA,
        pltpu.SemaphoreType.DMA,
    ]

    def body(feat, u0, u1, out, i0_v, i1_v, r0_v, r1_v, o_v, s0, s1):
        cid = lax.axis_index("c")
        sid = lax.axis_index("s")
        wid = sid * _NC + cid
        iota = lax.iota(_I32, 16)

        def chunk(j, _):
            b = wid * per_w + j * c_chunk
            pltpu.sync_copy(u0.at[pl.ds(b, c_chunk)], i0_v)
            pltpu.sync_copy(u1.at[pl.ds(b, c_chunk)], i1_v)
            cp0 = pltpu.async_copy(feat.at[i0_v], r0_v, s0)
            cp1 = pltpu.async_copy(feat.at[i1_v], r1_v, s1)
            cp0.wait()
            cp1.wait()

            def row(ei, _):
                ei16 = jnp.full((16,), ei, _I32)
                for q in range(fb):
                    a = plsc.load_gather(r0_v, [ei16, iota + q * 16])
                    bv = plsc.load_gather(r1_v, [ei16, iota + q * 16])
                    plsc.store_scatter(o_v, [ei16, iota + q * 16],
                                       (a + bv) * 0.5)
                return 0
            lax.fori_loop(0, c_chunk, row, 0)
            pltpu.sync_copy(o_v, out.at[pl.ds(b, c_chunk)])
            return 0
        lax.fori_loop(0, n_chunks, chunk, 0)

    return pl.kernel(
        body, out_type=jax.ShapeDtypeStruct((m_pad, f), _F32),
        scratch_types=scratch, mesh=_mesh(), compiler_params=_SC_PARAMS)


@functools.lru_cache(maxsize=None)
def _pool_call(num_pad):
    per_w = num_pad // _NW
    n_chunks = per_w // 16

    scratch = [
        pltpu.VMEM((112,), _I32),
        pltpu.VMEM((112, 16), _F32),
        pltpu.VMEM((16, 16), _F32),
        pltpu.SemaphoreType.DMA,
    ]

    def body(x, hexflat, out, h_v, r_v, o_v, sem):
        cid = lax.axis_index("c")
        sid = lax.axis_index("s")
        wid = sid * _NC + cid
        iota = lax.iota(_I32, 16)

        def chunk(j, _):
            b = wid * per_w + j * 16
            pltpu.sync_copy(hexflat.at[pl.ds(b * 7, 112)], h_v)
            pltpu.async_copy(x.at[h_v], r_v, sem).wait()
            for ei in range(16):
                acc = jnp.zeros((16,), _F32)
                for k in range(7):
                    pos = 7 * iota + k
                    acc = acc + plsc.load_gather(
                        r_v, [pos // 16 + 7 * ei, pos % 16])
                o_v[ei, :] = acc * (1.0 / 7.0)
            pltpu.sync_copy(o_v, out.at[pl.ds(b, 16)])
            return 0
        lax.fori_loop(0, n_chunks, chunk, 0)

    return pl.kernel(
        body, out_type=jax.ShapeDtypeStruct((num_pad, 16), _F32),
        scratch_types=scratch, mesh=_mesh(), compiler_params=_SC_PARAMS)


def _edge_chunk_size(e_pad):
    per_w = e_pad // _NW
    for c in (128, 120, 64, 32, 16, 8):
        if per_w % c == 0:
            return c
    return per_w


def _gmm_conv_sc(x, lvl, p, with_cnt, cnt=None):
    n = x.shape[0]
    cout = p["root"].shape[1]
    xg, xr = _matmul_tc(x, p)
    w = _gauss_w(lvl["pseudo"], p)
    call = _edge_call(lvl["e_pad"], n, lvl["n1"], cout, lvl["c_chunk"],
                      with_cnt)
    res = call(xg, w, lvl["src"], lvl["dst"])
    if with_cnt:
        agg, cnt = res
    else:
        agg = res[0] if isinstance(res, (list, tuple)) else res
    return agg, cnt, xr


def _res_block_sc(x, lvl, rp):
    n = x.shape[0]
    agg1, cnt, xr1 = _gmm_conv_sc(x, lvl, rp["conv1"], True)
    h = rp["conv1"]["root"].shape[1]
    bn = min(1024, _rup(n, 8))
    c0 = cnt[0, :n].reshape(n, 1)
    c1 = cnt[1, :n].reshape(n, 1)
    h1 = _comb1_call(n, h, bn, True)(
        agg1[0, :n], agg1[1, :n], c0, c1, xr1)

    agg2, _, xr2 = _gmm_conv_sc(h1, lvl, rp["conv2"], False)
    cout = rp["conv2"]["root"].shape[1]
    if "shortcut" in rp:
        aggs, _, xrs = _gmm_conv_sc(x, lvl, rp["shortcut"], False)
        return _comb2_call(n, cout, bn, False)(
            agg2[0, :n], agg2[1, :n], c0, c1, xr2,
            aggs[0, :n], aggs[1, :n], xrs)
    return _comb2_call(n, cout, bn, True)(
        agg2[0, :n], agg2[1, :n], c0, c1, xr2,
        agg2[0, :n], agg2[1, :n], x)


def _hex_up(feat, ups):
    m = ups.shape[0]
    m_pad = _rup(m, 256)
    u0 = jnp.pad(ups[:, 0], (0, m_pad - m))
    u1 = jnp.pad(ups[:, 1], (0, m_pad - m))
    per_w = m_pad // _NW
    c = per_w if per_w <= 128 else _edge_chunk_size(m_pad)
    new = _ups_call(m_pad, feat.shape[1], c)(feat, u0, u1)
    return jnp.concatenate([feat, new[:m]], axis=0)


def _hex_pl(x, hex_arr):
    num = (x.shape[0] + 6) // 4
    num_pad = _rup(num, 512)
    hf = jnp.pad(hex_arr[:num].reshape(-1), (0, (num_pad - num) * 7))
    out = _pool_call(num_pad)(x, hf)
    return out[:num]


def kernel(moving_img, target_img, feat_x, feat_y, params, edge_indexes,
           pseudos, hexes, upsamples):
    lvls = []
    for i, v in enumerate((40962, 10242, 2562, 642, 162)):
        e = 6 * (v - 2)
        e_pad = _rup(e, 256)
        ei = edge_indexes[i]
        src = jnp.pad(ei[0], (0, e_pad - e))
        dst = jnp.pad(ei[1], (0, e_pad - e), constant_values=v)
        psd = jnp.pad(pseudos[i], ((0, e_pad - e), (0, 0)))
        lvls.append({
            "src": src, "dst": dst, "pseudo": psd, "e_pad": e_pad,
            "n1": max(_rup(v + 1, 1024), 1024),
            "c_chunk": _edge_chunk_size(e_pad),
        })

    x = _head_tc(feat_x, feat_y)
    x = _res_block_sc(x, lvls[4], params["res1"])
    x = _hex_up(x, upsamples[3])
    x = _res_block_sc(x, lvls[3], params["res2"])
    x = _hex_up(x, upsamples[2])
    x = _res_block_sc(x, lvls[2], params["res3"])
    x = _hex_up(x, upsamples[1])
    x = _res_block_sc(x, lvls[1], params["res4"])
    x = _hex_up(x, upsamples[0])
    x = _res_block_sc(x, lvls[0], params["res5"])
    for i in range(4):
        x = _hex_pl(x, hexes[i])
    return _softmax_tc(x)


# speedup vs baseline: 1.0073x; 1.0073x over previous
import functools

import jax
import jax.numpy as jnp
from jax import lax
from jax.experimental import pallas as pl
from jax.experimental.pallas import tpu as pltpu
from jax.experimental.pallas import tpu_sc as plsc

_K = 10
_NC = 2
_NS = 16
_NW = _NC * _NS
_F32 = jnp.float32
_I32 = jnp.int32


def _cdiv(a, b):
    return (a + b - 1) // b


def _rup(a, b):
    return _cdiv(a, b) * b


_SC_PARAMS = pltpu.CompilerParams(
    needs_layout_passes=False, use_tc_tiling_on_sc=False)


def _mesh():
    return plsc.VectorSubcoreMesh(core_axis_name="c", subcore_axis_name="s")


@functools.lru_cache(maxsize=None)
def _matmul_call(n, cin, kc, cout, bn):
    def body(x_ref, g_ref, r_ref, b_ref, xg_ref, xr_ref):
        xb = x_ref[...]
        xg_ref[...] = jnp.dot(xb, g_ref[...], preferred_element_type=_F32)
        xr_ref[...] = jnp.dot(xb, r_ref[...], preferred_element_type=_F32) + b_ref[...]

    return pl.pallas_call(
        body,
        grid=(_cdiv(n, bn),),
        in_specs=[
            pl.BlockSpec((bn, cin), lambda i: (i, 0)),
            pl.BlockSpec((cin, kc), lambda i: (0, 0)),
            pl.BlockSpec((cin, cout), lambda i: (0, 0)),
            pl.BlockSpec((1, cout), lambda i: (0, 0)),
        ],
        out_specs=[
            pl.BlockSpec((bn, kc), lambda i: (i, 0)),
            pl.BlockSpec((bn, cout), lambda i: (i, 0)),
        ],
        out_shape=[
            jax.ShapeDtypeStruct((n, kc), _F32),
            jax.ShapeDtypeStruct((n, cout), _F32),
        ],
    )


def _matmul_tc(x, p):
    n, cin = x.shape
    kc = p["g"].shape[1]
    cout = p["root"].shape[1]
    bn = min(1024, _rup(n, 8))
    return _matmul_call(n, cin, kc, cout, bn)(
        x, p["g"], p["root"], p["bias"].reshape(1, cout))


@functools.lru_cache(maxsize=None)
def _gauss_call(e, be):
    def body(p_ref, m_ref, c_ref, w_ref):
        p0 = p_ref[:, 0:1]
        p1 = p_ref[:, 1:2]
        w_ref[...] = jnp.exp((p0 - m_ref[0:1, :]) ** 2 * c_ref[0:1, :]
                             + (p1 - m_ref[1:2, :]) ** 2 * c_ref[1:2, :])

    return pl.pallas_call(
        body,
        grid=(_cdiv(e, be),),
        in_specs=[
            pl.BlockSpec((be, 2), lambda i: (i, 0)),
            pl.BlockSpec((2, 16), lambda i: (0, 0)),
            pl.BlockSpec((2, 16), lambda i: (0, 0)),
        ],
        out_specs=pl.BlockSpec((be, 16), lambda i: (i, 0)),
        out_shape=jax.ShapeDtypeStruct((e, 16), _F32),
    )


def _gauss_w(pseudo, p):
    e = pseudo.shape[0]
    m = jnp.zeros((2, 16), _F32).at[:, :_K].set(p["mu"].T)
    c = jnp.zeros((2, 16), _F32).at[:, :_K].set(
        (-0.5 / (1e-15 + p["sigma"] * p["sigma"])).T)
    be = min(2048, _rup(e, 8))
    return _gauss_call(e, be)(pseudo, m, c)


@functools.lru_cache(maxsize=None)
def _comb1_call(n, cout, bn, relu):
    def body(a0, a1, c0, c1, xr, o):
        cnt = jnp.maximum(c0[...] + c1[...], 1.0)
        r = (a0[...] + a1[...]) / cnt + xr[...]
        o[...] = jnp.maximum(r, 0.0) if relu else r

    s2 = lambda: pl.BlockSpec((bn, cout), lambda i: (i, 0))
    s1 = lambda: pl.BlockSpec((bn, 1), lambda i: (i, 0))
    return pl.pallas_call(
        body,
        grid=(_cdiv(n, bn),),
        in_specs=[s2(), s2(), s1(), s1(), s2()],
        out_specs=s2(),
        out_shape=jax.ShapeDtypeStruct((n, cout), _F32),
    )


@functools.lru_cache(maxsize=None)
def _comb2_call(n, cout, bn, identity_shortcut):
    def body(a20, a21, c0, c1, xr2, s0, s1_, xrs, o):
        cnt = jnp.maximum(c0[...] + c1[...], 1.0)
        h2 = (a20[...] + a21[...]) / cnt + xr2[...]
        if identity_shortcut:
            xs = xrs[...]
        else:
            xs = (s0[...] + s1_[...]) / cnt + xrs[...]
        o[...] = jnp.maximum(h2 + xs, 0.0)

    s2 = lambda: pl.BlockSpec((bn, cout), lambda i: (i, 0))
    s1 = lambda: pl.BlockSpec((bn, 1), lambda i: (i, 0))
    return pl.pallas_call(
        body,
        grid=(_cdiv(n, bn),),
        in_specs=[s2(), s2(), s1(), s1(), s2(), s2(), s2(), s2()],
        out_specs=s2(),
        out_shape=jax.ShapeDtypeStruct((n, cout), _F32),
    )


def _head_tc(feat_x, feat_y):
    def body(fx_ref, fy_ref, o_ref):
        fx = fx_ref[...]
        fy = fy_ref[...]
        nx = jnp.sqrt(jnp.sum(fx * fx, axis=1, keepdims=True))
        ny = jnp.sqrt(jnp.sum(fy * fy, axis=1, keepdims=True))
        o_ref[:, :128] = fx / nx
        o_ref[:, 128:] = fy / ny

    return pl.pallas_call(
        body, out_shape=jax.ShapeDtypeStruct((162, 256), _F32)
    )(feat_x, feat_y)


def _softmax_tc(x):
    def body(x_ref, o_ref):
        v = x_ref[...]
        m = jnp.max(v, axis=1, keepdims=True)
        e = jnp.exp(v - m)
        o_ref[...] = e / jnp.sum(e, axis=1, keepdims=True)

    return pl.pallas_call(
        body, out_shape=jax.ShapeDtypeStruct(x.shape, x.dtype)
    )(x)


@functools.lru_cache(maxsize=None)
def _edge_call(e_pad, n, n1, cout, c_chunk, with_cnt):
    kc = _K * cout
    ncb = cout // 16
    eu = 4 if ncb <= 2 else 1
    per_w = e_pad // _NW
    n_chunks = per_w // c_chunk
    zrows = n1 // _NS
    zb = 64
    nz = zrows // zb

    out_type = [jax.ShapeDtypeStruct((_NC, n1, cout), _F32)]
    scratch = [
        pltpu.VMEM((c_chunk,), _I32),
        pltpu.VMEM((c_chunk,), _I32),
        pltpu.VMEM((c_chunk,), _I32),
        pltpu.VMEM((c_chunk, 16), _F32),
        pltpu.VMEM((c_chunk, kc), _F32),
        pltpu.VMEM((c_chunk, kc), _F32),
        pltpu.VMEM((c_chunk, cout), _F32),
        pltpu.VMEM((zb, cout), _F32),
        pltpu.VMEM_SHARED((n1, cout), _F32),
        pltpu.SemaphoreType.DMA,
        pltpu.SemaphoreType.DMA,
    ]
    if with_cnt:
        out_type.append(jax.ShapeDtypeStruct((_NC, n1), _F32))
        scratch += [
            pltpu.VMEM((_rup(c_chunk, 16),), _F32),
            pltpu.VMEM((zrows,), _F32),
            pltpu.VMEM_SHARED((n1,), _F32),
        ]

    def body(xg, w, src, dst, *rest):
        if with_cnt:
            (out, cnt_out, idx_v0, idx_v1, dst_v, w_v, rows_v0, rows_v1,
             msg_v, zb_v, agg_sh, sem0, sem1, ones_v, cz_v, cnt_sh) = rest
        else:
            (out, idx_v0, idx_v1, dst_v, w_v, rows_v0, rows_v1, msg_v, zb_v,
             agg_sh, sem0, sem1) = rest
        cid = lax.axis_index("c")
        sid = lax.axis_index("s")
        wid = sid * _NC + cid
        iota = lax.iota(_I32, 16)

        def fill_zb(i, _):
            for cb in range(ncb):
                zb_v[i, pl.ds(cb * 16, 16)] = jnp.zeros((16,), _F32)
            return 0
        lax.fori_loop(0, zb, fill_zb, 0)
        if with_cnt:
            def fill_ones(i, _):
                ones_v[pl.ds(i * 16, 16)] = jnp.ones((16,), _F32)
                return 0
            lax.fori_loop(0, _rup(c_chunk, 16) // 16, fill_ones, 0)

            def fill_cz(i, _):
                cz_v[pl.ds(i * 16, 16)] = jnp.zeros((16,), _F32)
                return 0
            lax.fori_loop(0, zrows // 16, fill_cz, 0)

        def zero_sh(j, _):
            pltpu.sync_copy(zb_v, agg_sh.at[pl.ds(sid * zrows + j * zb, zb)])
            return 0
        lax.fori_loop(0, nz, zero_sh, 0)
        if with_cnt:
            pltpu.sync_copy(cz_v, cnt_sh.at[pl.ds(sid * zrows, zrows)])
        plsc.subcore_barrier()

        base = wid * per_w

        def start_gather(j, idx_b, rows_b, sem_b):
            pltpu.sync_copy(src.at[pl.ds(base + j * c_chunk, c_chunk)], idx_b)
            pltpu.make_async_copy(xg.at[idx_b], rows_b, sem_b).start()

        def compute(j, idx_b, rows_b, sem_b):
            b = base + j * c_chunk
            pltpu.sync_copy(dst.at[pl.ds(b, c_chunk)], dst_v)
            pltpu.sync_copy(w.at[pl.ds(b, c_chunk)], w_v)
            pltpu.make_async_copy(xg.at[idx_b], rows_b, sem_b).wait()

            def edge(q, _):
                for u in range(eu):
                    ei = q * eu + u
                    ei16 = jnp.full((16,), ei, _I32)
                    accs = [jnp.zeros((16,), _F32) for _ in range(ncb)]
                    for k in range(_K):
                        wk = plsc.load_gather(
                            w_v, [ei16, jnp.full((16,), k, _I32)])
                        for cb in range(ncb):
                            r = plsc.load_gather(
                                rows_b, [ei16, iota + (k * cout + cb * 16)])
                            accs[cb] = accs[cb] + wk * r
                    for cb in range(ncb):
                        plsc.store_scatter(msg_v, [ei16, iota + cb * 16],
                                           accs[cb])
                return 0
            lax.fori_loop(0, c_chunk // eu, edge, 0)

            pltpu.sync_copy(msg_v, agg_sh.at[dst_v], add=True)
            if with_cnt:
                pltpu.sync_copy(ones_v.at[pl.ds(0, c_chunk)], cnt_sh.at[dst_v],
                                add=True)

        start_gather(0, idx_v0, rows_v0, sem0)

        def pair(jj, _):
            j0 = 2 * jj
            start_gather(j0 + 1, idx_v1, rows_v1, sem1)
            compute(j0, idx_v0, rows_v0, sem0)

            @pl.when(j0 + 2 < n_chunks)
            def _():
                start_gather(j0 + 2, idx_v0, rows_v0, sem0)
            compute(j0 + 1, idx_v1, rows_v1, sem1)
            return 0
        lax.fori_loop(0, n_chunks // 2, pair, 0)
        if n_chunks % 2 == 1:
            compute(n_chunks - 1, idx_v0, rows_v0, sem0)
        plsc.subcore_barrier()

        def drain(j, _):
            r0 = sid * zrows + j * zb
            pltpu.sync_copy(agg_sh.at[pl.ds(r0, zb)], zb_v)
            pltpu.sync_copy(zb_v, out.at[cid, pl.ds(r0, zb)])
            return 0
        lax.fori_loop(0, nz, drain, 0)
        if with_cnt:
            pltpu.sync_copy(cnt_sh.at[pl.ds(sid * zrows, zrows)], cz_v)
            pltpu.sync_copy(cz_v, cnt_out.at[cid, pl.ds(sid * zrows, zrows)])

    return pl.kernel(body, out_type=out_type, scratch_types=scratch,
                     mesh=_mesh(), compiler_params=_SC_PARAMS)


@functools.lru_cache(maxsize=None)
def _ups_call(m_pad, f, c_chunk):
    per_w = m_pad // _NW
    n_chunks = per_w // c_chunk
    fb = f // 16

    scratch = [
        pltpu.VMEM((c_chunk,), _I32),
        pltpu.VMEM((c_chunk,), _I32),
        pltpu.VMEM((c_chunk, f), _F32),
        pltpu.VMEM((c_chunk, f), _F32),
        pltpu.VMEM((c_chunk, f), _F32),
        pltpu.SemaphoreType.DMA,
        pltpu.SemaphoreType.DMA,
    ]

    def body(feat, u0, u1, out, i0_v, i1_v, r0_v, r1_v, o_v, s0, s1):
        cid = lax.axis_index("c")
        sid = lax.axis_index("s")
        wid = sid * _NC + cid
        iota = lax.iota(_I32, 16)

        def chunk(j, _):
            b = wid * per_w + j * c_chunk
            pltpu.sync_copy(u0.at[pl.ds(b, c_chunk)], i0_v)
            pltpu.sync_copy(u1.at[pl.ds(b, c_chunk)], i1_v)
            cp0 = pltpu.async_copy(feat.at[i0_v], r0_v, s0)
            cp1 = pltpu.async_copy(feat.at[i1_v], r1_v, s1)
            cp0.wait()
            cp1.wait()

            def row(ei, _):
                ei16 = jnp.full((16,), ei, _I32)
                for q in range(fb):
                    a = plsc.load_gather(r0_v, [ei16, iota + q * 16])
                    bv = plsc.load_gather(r1_v, [ei16, iota + q * 16])
                    plsc.store_scatter(o_v, [ei16, iota + q * 16],
                                       (a + bv) * 0.5)
                return 0
            lax.fori_loop(0, c_chunk, row, 0)
            pltpu.sync_copy(o_v, out.at[pl.ds(b, c_chunk)])
            return 0
        lax.fori_loop(0, n_chunks, chunk, 0)

    return pl.kernel(
        body, out_type=jax.ShapeDtypeStruct((m_pad, f), _F32),
        scratch_types=scratch, mesh=_mesh(), compiler_params=_SC_PARAMS)


@functools.lru_cache(maxsize=None)
def _pool_call(num_pad):
    per_w = num_pad // _NW
    n_chunks = per_w // 16

    scratch = [
        pltpu.VMEM((112,), _I32),
        pltpu.VMEM((112, 16), _F32),
        pltpu.VMEM((16, 16), _F32),
        pltpu.SemaphoreType.DMA,
    ]

    def body(x, hexflat, out, h_v, r_v, o_v, sem):
        cid = lax.axis_index("c")
        sid = lax.axis_index("s")
        wid = sid * _NC + cid
        iota = lax.iota(_I32, 16)

        def chunk(j, _):
            b = wid * per_w + j * 16
            pltpu.sync_copy(hexflat.at[pl.ds(b * 7, 112)], h_v)
            pltpu.async_copy(x.at[h_v], r_v, sem).wait()
            for ei in range(16):
                acc = jnp.zeros((16,), _F32)
                for k in range(7):
                    pos = 7 * iota + k
                    acc = acc + plsc.load_gather(
                        r_v, [pos // 16 + 7 * ei, pos % 16])
                o_v[ei, :] = acc * (1.0 / 7.0)
            pltpu.sync_copy(o_v, out.at[pl.ds(b, 16)])
            return 0
        lax.fori_loop(0, n_chunks, chunk, 0)

    return pl.kernel(
        body, out_type=jax.ShapeDtypeStruct((num_pad, 16), _F32),
        scratch_types=scratch, mesh=_mesh(), compiler_params=_SC_PARAMS)


@functools.lru_cache(maxsize=None)
def _poolchain_call(v0):
    pads = []
    n = v0
    for _ in range(4):
        n = (n + 6) // 4
        pads.append(_rup(n, 512))
    np0, np1, np2, np3 = pads

    scratch = [
        pltpu.VMEM((112,), _I32),
        pltpu.VMEM((112, 16), _F32),
        pltpu.VMEM((16, 16), _F32),
        pltpu.VMEM_SHARED((np0, 16), _F32),
        pltpu.VMEM_SHARED((np1, 16), _F32),
        pltpu.VMEM_SHARED((np2, 16), _F32),
        pltpu.SemaphoreType.DMA,
    ]

    def body(x, hf0, hf1, hf2, hf3, out, h_v, r_v, o_v, st0, st1, st2, sem):
        cid = lax.axis_index("c")
        sid = lax.axis_index("s")
        iota = lax.iota(_I32, 16)

        @pl.when(cid == 0)
        def _():
            def stage(src_ref, hf, num_pad, dst_ref):
                per_t = num_pad // _NS

                def chunk(j, _):
                    b = sid * per_t + j * 16
                    pltpu.sync_copy(hf.at[pl.ds(b * 7, 112)], h_v)
                    pltpu.async_copy(src_ref.at[h_v], r_v, sem).wait()
                    for ei in range(16):
                        acc = jnp.zeros((16,), _F32)
                        for k in range(7):
                            pos = 7 * iota + k
                            acc = acc + plsc.load_gather(
                                r_v, [pos // 16 + 7 * ei, pos % 16])
                        o_v[ei, :] = acc * (1.0 / 7.0)
                    pltpu.sync_copy(o_v, dst_ref.at[pl.ds(b, 16)])
                    return 0
                lax.fori_loop(0, per_t // 16, chunk, 0)

            stage(x, hf0, np0, st0)
            plsc.subcore_barrier()
            stage(st0, hf1, np1, st1)
            plsc.subcore_barrier()
            stage(st1, hf2, np2, st2)
            plsc.subcore_barrier()
            stage(st2, hf3, np3, out)

    return pl.kernel(
        body, out_type=jax.ShapeDtypeStruct((np3, 16), _F32),
        scratch_types=scratch, mesh=_mesh(), compiler_params=_SC_PARAMS)


def _edge_chunk_size(e_pad):
    per_w = e_pad // _NW
    for c in (128, 120, 64, 32, 16, 8):
        if per_w % c == 0:
            return c
    return per_w


def _gmm_conv_sc(x, lvl, p, with_cnt, cnt=None):
    n = x.shape[0]
    cout = p["root"].shape[1]
    xg, xr = _matmul_tc(x, p)
    w = _gauss_w(lvl["pseudo"], p)
    call = _edge_call(lvl["e_pad"], n, lvl["n1"], cout, lvl["c_chunk"],
                      with_cnt)
    res = call(xg, w, lvl["src"], lvl["dst"])
    if with_cnt:
        agg, cnt = res
    else:
        agg = res[0] if isinstance(res, (list, tuple)) else res
    return agg, cnt, xr


def _res_block_sc(x, lvl, rp):
    n = x.shape[0]
    agg1, cnt, xr1 = _gmm_conv_sc(x, lvl, rp["conv1"], True)
    h = rp["conv1"]["root"].shape[1]
    bn = min(1024, _rup(n, 8))
    c0 = cnt[0, :n].reshape(n, 1)
    c1 = cnt[1, :n].reshape(n, 1)
    h1 = _comb1_call(n, h, bn, True)(
        agg1[0, :n], agg1[1, :n], c0, c1, xr1)

    agg2, _, xr2 = _gmm_conv_sc(h1, lvl, rp["conv2"], False)
    cout = rp["conv2"]["root"].shape[1]
    if "shortcut" in rp:
        aggs, _, xrs = _gmm_conv_sc(x, lvl, rp["shortcut"], False)
        return _comb2_call(n, cout, bn, False)(
            agg2[0, :n], agg2[1, :n], c0, c1, xr2,
            aggs[0, :n], aggs[1, :n], xrs)
    return _comb2_call(n, cout, bn, True)(
        agg2[0, :n], agg2[1, :n], c0, c1, xr2,
        agg2[0, :n], agg2[1, :n], x)


def _hex_up(feat, ups):
    m = ups.shape[0]
    m_pad = _rup(m, 256)
    u0 = jnp.pad(ups[:, 0], (0, m_pad - m))
    u1 = jnp.pad(ups[:, 1], (0, m_pad - m))
    per_w = m_pad // _NW
    c = per_w if per_w <= 128 else _edge_chunk_size(m_pad)
    new = _ups_call(m_pad, feat.shape[1], c)(feat, u0, u1)
    return jnp.concatenate([feat, new[:m]], axis=0)


def _hex_pl(x, hex_arr):
    num = (x.shape[0] + 6) // 4
    num_pad = _rup(num, 512)
    hf = jnp.pad(hex_arr[:num].reshape(-1), (0, (num_pad - num) * 7))
    out = _pool_call(num_pad)(x, hf)
    return out[:num]


def kernel(moving_img, target_img, feat_x, feat_y, params, edge_indexes,
           pseudos, hexes, upsamples):
    lvls = []
    for i, v in enumerate((40962, 10242, 2562, 642, 162)):
        e = 6 * (v - 2)
        e_pad = _rup(e, 256)
        ei = edge_indexes[i]
        src = jnp.pad(ei[0], (0, e_pad - e))
        dst = jnp.pad(ei[1], (0, e_pad - e), constant_values=v)
        psd = jnp.pad(pseudos[i], ((0, e_pad - e), (0, 0)))
        lvls.append({
            "src": src, "dst": dst, "pseudo": psd, "e_pad": e_pad,
            "n1": max(_rup(v + 1, 1024), 1024),
            "c_chunk": _edge_chunk_size(e_pad),
        })

    x = _head_tc(feat_x, feat_y)
    x = _res_block_sc(x, lvls[4], params["res1"])
    x = _hex_up(x, upsamples[3])
    x = _res_block_sc(x, lvls[3], params["res2"])
    x = _hex_up(x, upsamples[2])
    x = _res_block_sc(x, lvls[2], params["res3"])
    x = _hex_up(x, upsamples[1])
    x = _res_block_sc(x, lvls[1], params["res4"])
    x = _hex_up(x, upsamples[0])
    x = _res_block_sc(x, lvls[0], params["res5"])
    hfs = []
    n_cur = 40962
    for i in range(4):
        num = (n_cur + 6) // 4
        num_pad = _rup(num, 512)
        hfs.append(jnp.pad(hexes[i][:num].reshape(-1),
                           (0, (num_pad - num) * 7)))
        n_cur = num
    x = _poolchain_call(40962)(x, hfs[0], hfs[1], hfs[2], hfs[3])[:162]
    return _softmax_tc(x)
